# baseline (speedup 1.0000x reference)
import functools
import math

import jax
import jax.numpy as jnp
from jax import lax
from jax.experimental import pallas as pl
from jax.experimental.pallas import tpu as pltpu
from jax.experimental.pallas import tpu_sc as plsc

D_MODEL = 64
LANES = 16
CHUNK = 100
NBUF = 4


@functools.cache
def _build(B, S, V):
    info = plsc.get_sparse_core_info()
    nc, ns = info.num_cores, info.num_subcores
    nw = nc * ns
    n = B * S
    rows_w = n // nw
    nch = rows_w // CHUNK
    period = S // CHUNK
    scale = jnp.float32(math.sqrt(D_MODEL))
    assert n % nw == 0 and rows_w % CHUNK == 0 and S % CHUNK == 0
    assert nch % NBUF == 0

    mesh = plsc.VectorSubcoreMesh(core_axis_name="c", subcore_axis_name="s")

    @functools.partial(
        pl.kernel,
        mesh=mesh,
        compiler_params=pltpu.CompilerParams(use_tc_tiling_on_sc=False),
        out_type=jax.ShapeDtypeStruct((nw, nch, CHUNK, D_MODEL), jnp.float32),
        scratch_types=(
            [pltpu.VMEM((nch, CHUNK), jnp.int32),
             pltpu.VMEM((S, D_MODEL), jnp.float32)]
            + [pltpu.VMEM((CHUNK, D_MODEL), jnp.float32) for _ in range(NBUF)]
            + [pltpu.SemaphoreType.DMA for _ in range(2 * NBUF)]
        ),
    )
    def kern(idx_hbm, pe_hbm, table_hbm, out_hbm, idx_v, pe_v, *rest):
        bufs = rest[:NBUF]
        gsem = rest[NBUF:2 * NBUF]
        ssem = rest[2 * NBUF:]
        wid = lax.axis_index("s") * nc + lax.axis_index("c")

        pltpu.sync_copy(idx_hbm.at[wid], idx_v)
        pltpu.sync_copy(pe_hbm, pe_v)

        def start_gather(k, b):
            pltpu.async_copy(table_hbm.at[idx_v.at[k]], bufs[b], gsem[b])

        def wait_gather(k, b):
            pltpu.make_async_copy(table_hbm.at[idx_v.at[k]], bufs[b],
                                  gsem[b]).wait()

        def start_scatter(k, b):
            pltpu.async_copy(bufs[b], out_hbm.at[wid, k], ssem[b])

        def wait_scatter(k, b):
            pltpu.make_async_copy(bufs[b], out_hbm.at[wid, k], ssem[b]).wait()

        def compute(k, b):
            s0 = (k % period) * CHUNK

            def row(r, carry):
                for j in range(D_MODEL // LANES):
                    sl = pl.ds(j * LANES, LANES)
                    bufs[b][r, sl] = (bufs[b][r, sl] * scale
                                      + pe_v[s0 + r, sl])
                return carry

            lax.fori_loop(0, CHUNK, row, 0)

        for b in range(NBUF):
            start_gather(b, b)

        def outer(i, carry):
            for b in range(NBUF):
                k = i * NBUF + b
                wait_gather(k, b)
                compute(k, b)
                start_scatter(k, b)
                kp = k + NBUF - 1
                bp = (b - 1) % NBUF

                @pl.when((k >= 1) & (kp < nch))
                def _():
                    wait_scatter(k - 1, bp)
                    start_gather(kp, bp)
            return carry

        lax.fori_loop(0, nch // NBUF, outer, 0)

        for b in range(NBUF):
            wait_scatter(nch - NBUF + b, b)

    return kern, nw, nch


def kernel(x, table, pe):
    B, S = x.shape
    V, D = table.shape
    kern, nw, nch = _build(B, S, V)
    idx = x.astype(jnp.int32).reshape(nw, nch, CHUNK)
    pe2 = pe[:S, 0, :]
    out = kern(idx, pe2, table)
    return out.reshape(B, S, D)


# speedup vs baseline: 1.0001x; 1.0001x over previous
import functools
import math

import jax
import jax.numpy as jnp
from jax import lax
from jax.experimental import pallas as pl
from jax.experimental.pallas import tpu as pltpu
from jax.experimental.pallas import tpu_sc as plsc

D_MODEL = 64
LANES = 16
CHUNK = 100
NBUF = 4


@functools.cache
def _build(B, S, V):
    info = plsc.get_sparse_core_info()
    nc, ns = info.num_cores, info.num_subcores
    nw = nc * ns
    n = B * S
    rows_w = n // nw
    nch = rows_w // CHUNK
    period = S // CHUNK
    scale = jnp.float32(math.sqrt(D_MODEL))
    assert n % nw == 0 and rows_w % CHUNK == 0 and S % CHUNK == 0
    assert nch % NBUF == 0

    mesh = plsc.VectorSubcoreMesh(core_axis_name="c", subcore_axis_name="s")

    @functools.partial(
        pl.kernel,
        mesh=mesh,
        compiler_params=pltpu.CompilerParams(use_tc_tiling_on_sc=False),
        out_type=jax.ShapeDtypeStruct((B, S, D_MODEL), jnp.float32),
        scratch_types=(
            [pltpu.VMEM((nch, CHUNK), jnp.int32),
             pltpu.VMEM((S, D_MODEL), jnp.float32)]
            + [pltpu.VMEM((CHUNK, D_MODEL), jnp.float32) for _ in range(NBUF)]
            + [pltpu.SemaphoreType.DMA for _ in range(2 * NBUF)]
        ),
    )
    def kern(idx_hbm, pe_hbm, table_hbm, out_hbm, idx_v, pe_v, *rest):
        bufs = rest[:NBUF]
        gsem = rest[NBUF:2 * NBUF]
        ssem = rest[2 * NBUF:]
        wid = lax.axis_index("s") * nc + lax.axis_index("c")

        pltpu.sync_copy(idx_hbm.at[wid], idx_v)
        pltpu.sync_copy(pe_hbm, pe_v)

        def start_gather(k, b):
            pltpu.async_copy(table_hbm.at[idx_v.at[k]], bufs[b], gsem[b])

        def wait_gather(k, b):
            pltpu.make_async_copy(table_hbm.at[idx_v.at[k]], bufs[b],
                                  gsem[b]).wait()

        def _out_dst(k):
            bi = wid * (rows_w // S) + k // (S // CHUNK)
            s0 = (k % (S // CHUNK)) * CHUNK
            return out_hbm.at[bi, pl.ds(s0, CHUNK)]

        def start_scatter(k, b):
            pltpu.async_copy(bufs[b], _out_dst(k), ssem[b])

        def wait_scatter(k, b):
            pltpu.make_async_copy(bufs[b], _out_dst(k), ssem[b]).wait()

        def compute(k, b):
            s0 = (k % period) * CHUNK

            def row(r, carry):
                for j in range(D_MODEL // LANES):
                    sl = pl.ds(j * LANES, LANES)
                    bufs[b][r, sl] = (bufs[b][r, sl] * scale
                                      + pe_v[s0 + r, sl])
                return carry

            lax.fori_loop(0, CHUNK, row, 0)

        for b in range(NBUF):
            start_gather(b, b)

        def outer(i, carry):
            for b in range(NBUF):
                k = i * NBUF + b
                wait_gather(k, b)
                compute(k, b)
                start_scatter(k, b)
                kp = k + NBUF - 1
                bp = (b - 1) % NBUF

                @pl.when((k >= 1) & (kp < nch))
                def _():
                    wait_scatter(k - 1, bp)
                    start_gather(kp, bp)
            return carry

        lax.fori_loop(0, nch // NBUF, outer, 0)

        for b in range(NBUF):
            wait_scatter(nch - NBUF + b, b)

    return kern, nw, nch


def kernel(x, table, pe):
    B, S = x.shape
    V, D = table.shape
    kern, nw, nch = _build(B, S, V)
    idx = x.astype(jnp.int32).reshape(nw, nch, CHUNK)
    pe2 = pe[:S, 0, :]
    return kern(idx, pe2, table)
